# baseline (device time: 103919 ns/iter reference)
import jax
import jax.numpy as jnp
from jax import lax
from jax.experimental import pallas as pl
from jax.experimental.pallas import tpu as pltpu

N_DEV = 8
E_LOCAL = 4
N_TOK = 512
D = 256
H = 512
N_EXP = 32


def kernel(x, router_W, route_idx, expert_W, shared_W):
    def body(x_ref, rw_ref, idx_ref, ew_ref, sw_ref, out_ref,
             comm_ref, send_sems, recv_sems):
        my_i = lax.axis_index("i")
        left = lax.rem(my_i - 1 + N_DEV, N_DEV)
        right = lax.rem(my_i + 1, N_DEV)

        barrier_sem = pltpu.get_barrier_semaphore()
        for nbr in (left, right):
            pl.semaphore_signal(
                barrier_sem, inc=1,
                device_id=(nbr,), device_id_type=pl.DeviceIdType.MESH,
            )
        pl.semaphore_wait(barrier_sem, 2)

        x = x_ref[...]
        idx = idx_ref[...]

        scores = jnp.dot(x, rw_ref[...], preferred_element_type=jnp.float32)
        s_max = jnp.max(scores, axis=1, keepdims=True)
        p = jnp.exp(scores - s_max)
        probs = p / jnp.sum(p, axis=1, keepdims=True)

        cols = lax.broadcasted_iota(jnp.int32, (N_TOK, N_EXP), 1)
        sel_p = jnp.sum(jnp.where(cols == idx, probs, 0.0), axis=1,
                        keepdims=True)

        partial = jnp.zeros((N_TOK, H), dtype=jnp.float32)
        for k in range(E_LOCAL):
            e_k = my_i * E_LOCAL + k
            coeff = jnp.where(idx == e_k, sel_p, 0.0)
            partial = partial + jnp.dot(
                x * coeff, ew_ref[k], preferred_element_type=jnp.float32
            )

        comm_ref[0] = partial
        out_ref[...] = partial + jnp.dot(
            x, sw_ref[...], preferred_element_type=jnp.float32
        )

        for h in range(N_DEV - 1):
            rdma = pltpu.make_async_remote_copy(
                src_ref=comm_ref.at[h],
                dst_ref=comm_ref.at[h + 1],
                send_sem=send_sems.at[h],
                recv_sem=recv_sems.at[h],
                device_id=(right,),
                device_id_type=pl.DeviceIdType.MESH,
            )
            rdma.start()
            rdma.wait()
            out_ref[...] = out_ref[...] + comm_ref[h + 1]

    return pl.pallas_call(
        body,
        out_shape=jax.ShapeDtypeStruct((N_TOK, H), jnp.float32),
        in_specs=[pl.BlockSpec(memory_space=pltpu.VMEM)] * 5,
        out_specs=pl.BlockSpec(memory_space=pltpu.VMEM),
        scratch_shapes=[
            pltpu.VMEM((N_DEV, N_TOK, H), jnp.float32),
            pltpu.SemaphoreType.DMA((N_DEV - 1,)),
            pltpu.SemaphoreType.DMA((N_DEV - 1,)),
        ],
        compiler_params=pltpu.CompilerParams(collective_id=0),
    )(x, router_W, route_idx, expert_W, shared_W)


# device time: 41281 ns/iter; 2.5174x vs baseline; 2.5174x over previous
import jax
import jax.numpy as jnp
from jax import lax
from jax.experimental import pallas as pl
from jax.experimental.pallas import tpu as pltpu

N_DEV = 8
E_LOCAL = 4
N_TOK = 512
D = 256
H = 512
N_EXP = 32


def kernel(x, router_W, route_idx, expert_W, shared_W):
    def body(x_ref, rw_ref, idx_ref, ew_ref, sw_ref, out_ref,
             acc_ref, st0, st1, st2, rs_send, rs_recv, ag_send, ag_recv):
        my_i = lax.axis_index("i")
        l = lax.rem(my_i, 4)
        zb = (my_i >= 4).astype(jnp.int32)
        xb = ((l == 1) | (l == 2)).astype(jnp.int32)
        yb = (l >= 2).astype(jnp.int32)
        p_z = lax.rem(my_i + 4, N_DEV)
        p_x = my_i + 1 - 2 * lax.rem(my_i, 2)
        p_y = my_i + 3 - 2 * l

        barrier_sem = pltpu.get_barrier_semaphore()
        for nbr in (p_x, p_y, p_z):
            pl.semaphore_signal(
                barrier_sem, inc=1,
                device_id=(nbr,), device_id_type=pl.DeviceIdType.MESH,
            )
        pl.semaphore_wait(barrier_sem, 3)

        x = x_ref[...]
        idx = idx_ref[...]

        scores = jnp.dot(x, rw_ref[...], preferred_element_type=jnp.float32)
        s_max = jnp.max(scores, axis=1, keepdims=True)
        p = jnp.exp(scores - s_max)
        probs = p / jnp.sum(p, axis=1, keepdims=True)

        cols = lax.broadcasted_iota(jnp.int32, (N_TOK, N_EXP), 1)
        sel_p = jnp.sum(jnp.where(cols == idx, probs, 0.0), axis=1,
                        keepdims=True)

        partial = jnp.dot(x * 0.125, sw_ref[...],
                          preferred_element_type=jnp.float32)
        for k in range(E_LOCAL):
            e_k = my_i * E_LOCAL + k
            coeff = jnp.where(idx == e_k, sel_p, 0.0)
            partial = partial + jnp.dot(
                x * coeff, ew_ref[k], preferred_element_type=jnp.float32
            )
        acc_ref[...] = partial

        keep = jnp.int32(0)
        for s, (size, part, bit, st) in enumerate((
            (256, p_z, zb, st0),
            (128, p_x, xb, st1),
            (64, p_y, yb, st2),
        )):
            send_start = keep + (1 - bit) * size
            keep = keep + bit * size
            rdma = pltpu.make_async_remote_copy(
                src_ref=acc_ref.at[pl.ds(send_start, size)],
                dst_ref=st,
                send_sem=rs_send.at[s],
                recv_sem=rs_recv.at[s],
                device_id=(part,),
                device_id_type=pl.DeviceIdType.MESH,
            )
            rdma.start()
            rdma.wait()
            acc_ref[pl.ds(keep, size)] = acc_ref[pl.ds(keep, size)] + st[...]

        out_ref[pl.ds(keep, 64)] = acc_ref[pl.ds(keep, 64)]

        own = keep
        for s, (size, part, bit) in enumerate((
            (64, p_y, yb),
            (128, p_x, xb),
            (256, p_z, zb),
        )):
            rdma = pltpu.make_async_remote_copy(
                src_ref=out_ref.at[pl.ds(own, size)],
                dst_ref=out_ref.at[pl.ds(own, size)],
                send_sem=ag_send.at[s],
                recv_sem=ag_recv.at[s],
                device_id=(part,),
                device_id_type=pl.DeviceIdType.MESH,
            )
            rdma.start()
            rdma.wait()
            own = own - bit * size

    return pl.pallas_call(
        body,
        out_shape=jax.ShapeDtypeStruct((N_TOK, H), jnp.float32),
        in_specs=[pl.BlockSpec(memory_space=pltpu.VMEM)] * 5,
        out_specs=pl.BlockSpec(memory_space=pltpu.VMEM),
        scratch_shapes=[
            pltpu.VMEM((N_TOK, H), jnp.float32),
            pltpu.VMEM((256, H), jnp.float32),
            pltpu.VMEM((128, H), jnp.float32),
            pltpu.VMEM((64, H), jnp.float32),
            pltpu.SemaphoreType.DMA((3,)),
            pltpu.SemaphoreType.DMA((3,)),
            pltpu.SemaphoreType.DMA((3,)),
            pltpu.SemaphoreType.DMA((3,)),
        ],
        compiler_params=pltpu.CompilerParams(collective_id=0),
    )(x, router_W, route_idx, expert_W, shared_W)


# device time: 23310 ns/iter; 4.4581x vs baseline; 1.7710x over previous
import jax
import jax.numpy as jnp
from jax import lax
from jax.experimental import pallas as pl
from jax.experimental.pallas import tpu as pltpu

N_DEV = 8
E_LOCAL = 4
N_TOK = 512
D = 256
H = 512
N_EXP = 32

PARTS = ((0, 176), (176, 176), (352, 160))


def kernel(x, router_W, route_idx, expert_W, shared_W):
    def body(x_ref, rw_ref, idx_ref, ew_ref, sw_ref, out_ref,
             acc_ref, st0, st1, st2, send_sems, recv_sems):
        my_i = lax.axis_index("i")
        l = lax.rem(my_i, 4)
        p_z = lax.rem(my_i + 4, N_DEV)
        p_x = my_i + 1 - 2 * lax.rem(my_i, 2)
        p_y = my_i + 3 - 2 * l

        barrier_sem = pltpu.get_barrier_semaphore()
        for nbr in (p_x, p_y, p_z):
            pl.semaphore_signal(
                barrier_sem, inc=1,
                device_id=(nbr,), device_id_type=pl.DeviceIdType.MESH,
            )
        pl.semaphore_wait(barrier_sem, 3)

        x = x_ref[...]
        idx = idx_ref[...]

        scores = jnp.dot(x, rw_ref[...], preferred_element_type=jnp.float32)
        s_max = jnp.max(scores, axis=1, keepdims=True)
        p = jnp.exp(scores - s_max)
        probs = p / jnp.sum(p, axis=1, keepdims=True)

        cols = lax.broadcasted_iota(jnp.int32, (N_TOK, N_EXP), 1)
        sel_p = jnp.sum(jnp.where(cols == idx, probs, 0.0), axis=1,
                        keepdims=True)

        x16 = x.astype(jnp.bfloat16)
        partial = jnp.dot((x * 0.125).astype(jnp.bfloat16),
                          sw_ref[...].astype(jnp.bfloat16),
                          preferred_element_type=jnp.float32)
        for k in range(E_LOCAL):
            e_k = my_i * E_LOCAL + k
            coeff = jnp.where(idx == e_k, sel_p, 0.0)
            partial = partial + jnp.dot(
                (x16 * coeff.astype(jnp.bfloat16)),
                ew_ref[k].astype(jnp.bfloat16),
                preferred_element_type=jnp.float32,
            )
        acc_ref[...] = partial.astype(jnp.bfloat16)

        orders = (
            (p_z, p_x, p_y),
            (p_x, p_y, p_z),
            (p_y, p_z, p_x),
        )
        stages = (st0, st1, st2)
        for s in range(3):
            rdmas = []
            for pi, (start, n) in enumerate(PARTS):
                rdma = pltpu.make_async_remote_copy(
                    src_ref=acc_ref.at[pl.ds(start, n)],
                    dst_ref=stages[pi].at[s],
                    send_sem=send_sems.at[pi, s],
                    recv_sem=recv_sems.at[pi, s],
                    device_id=(orders[pi][s],),
                    device_id_type=pl.DeviceIdType.MESH,
                )
                rdma.start()
                rdmas.append(rdma)
            for pi, (start, n) in enumerate(PARTS):
                rdmas[pi].wait_recv()
                acc_ref[pl.ds(start, n)] = (
                    acc_ref[pl.ds(start, n)] + stages[pi][s, :n]
                )
            for pi in range(3):
                rdmas[pi].wait_send()

        out_ref[...] = acc_ref[...].astype(jnp.float32)

    return pl.pallas_call(
        body,
        out_shape=jax.ShapeDtypeStruct((N_TOK, H), jnp.float32),
        in_specs=[pl.BlockSpec(memory_space=pltpu.VMEM)] * 5,
        out_specs=pl.BlockSpec(memory_space=pltpu.VMEM),
        scratch_shapes=[
            pltpu.VMEM((N_TOK, H), jnp.bfloat16),
            pltpu.VMEM((3, 176, H), jnp.bfloat16),
            pltpu.VMEM((3, 176, H), jnp.bfloat16),
            pltpu.VMEM((3, 160, H), jnp.bfloat16),
            pltpu.SemaphoreType.DMA((3, 3)),
            pltpu.SemaphoreType.DMA((3, 3)),
        ],
        compiler_params=pltpu.CompilerParams(collective_id=0),
    )(x, router_W, route_idx, expert_W, shared_W)


# device time: 23131 ns/iter; 4.4926x vs baseline; 1.0077x over previous
import jax
import jax.numpy as jnp
from jax import lax
from jax.experimental import pallas as pl
from jax.experimental.pallas import tpu as pltpu

N_DEV = 8
E_LOCAL = 4
N_TOK = 512
D = 256
H = 512
N_EXP = 32

PARTS = ((0, 176), (176, 176), (352, 160))


def kernel(x, router_W, route_idx, expert_W, shared_W):
    def body(x_ref, rw_ref, idx_ref, ew_ref, sw_ref, out_ref,
             acc_ref, st0, st1, st2, send_sems, recv_sems):
        my_i = lax.axis_index("i")
        l = lax.rem(my_i, 4)
        p_z = lax.rem(my_i + 4, N_DEV)
        p_x = my_i + 1 - 2 * lax.rem(my_i, 2)
        p_y = my_i + 3 - 2 * l

        barrier_sem = pltpu.get_barrier_semaphore()
        for nbr in (p_x, p_y, p_z):
            pl.semaphore_signal(
                barrier_sem, inc=1,
                device_id=(nbr,), device_id_type=pl.DeviceIdType.MESH,
            )
        pl.semaphore_wait(barrier_sem, 3)

        x = x_ref[...]
        idx = idx_ref[...]

        scores = jnp.dot(x, rw_ref[...], preferred_element_type=jnp.float32)
        s_max = jnp.max(scores, axis=1, keepdims=True)
        p = jnp.exp(scores - s_max)
        probs = p / jnp.sum(p, axis=1, keepdims=True)

        cols = lax.broadcasted_iota(jnp.int32, (N_TOK, N_EXP), 1)
        sel_p = jnp.sum(jnp.where(cols == idx, probs, 0.0), axis=1,
                        keepdims=True)

        coeffs = [jnp.full((N_TOK, 1), 0.125, jnp.float32)]
        for k in range(E_LOCAL):
            e_k = my_i * E_LOCAL + k
            coeffs.append(jnp.where(idx == e_k, sel_p, 0.0))
        xs = jnp.concatenate(
            [(x * c).astype(jnp.bfloat16) for c in coeffs], axis=1
        )
        W5 = jnp.concatenate(
            [sw_ref[...], ew_ref[...].reshape(E_LOCAL * D, H)], axis=0
        ).astype(jnp.bfloat16)
        partial = jnp.dot(xs, W5, preferred_element_type=jnp.float32)
        acc_ref[...] = partial.astype(jnp.bfloat16)

        orders = (
            (p_z, p_x, p_y),
            (p_x, p_y, p_z),
            (p_y, p_z, p_x),
        )
        stages = (st0, st1, st2)
        for s in range(3):
            rdmas = []
            for pi, (start, n) in enumerate(PARTS):
                rdma = pltpu.make_async_remote_copy(
                    src_ref=acc_ref.at[pl.ds(start, n)],
                    dst_ref=stages[pi].at[s],
                    send_sem=send_sems.at[pi, s],
                    recv_sem=recv_sems.at[pi, s],
                    device_id=(orders[pi][s],),
                    device_id_type=pl.DeviceIdType.MESH,
                )
                rdma.start()
                rdmas.append(rdma)
            for pi, (start, n) in enumerate(PARTS):
                rdmas[pi].wait_recv()
                acc_ref[pl.ds(start, n)] = (
                    acc_ref[pl.ds(start, n)] + stages[pi][s, :n]
                )
            for pi in range(3):
                rdmas[pi].wait_send()

        out_ref[...] = acc_ref[...].astype(jnp.float32)

    return pl.pallas_call(
        body,
        out_shape=jax.ShapeDtypeStruct((N_TOK, H), jnp.float32),
        in_specs=[pl.BlockSpec(memory_space=pltpu.VMEM)] * 5,
        out_specs=pl.BlockSpec(memory_space=pltpu.VMEM),
        scratch_shapes=[
            pltpu.VMEM((N_TOK, H), jnp.bfloat16),
            pltpu.VMEM((3, 176, H), jnp.bfloat16),
            pltpu.VMEM((3, 176, H), jnp.bfloat16),
            pltpu.VMEM((3, 160, H), jnp.bfloat16),
            pltpu.SemaphoreType.DMA((3, 3)),
            pltpu.SemaphoreType.DMA((3, 3)),
        ],
        compiler_params=pltpu.CompilerParams(collective_id=0),
    )(x, router_W, route_idx, expert_W, shared_W)
